# baseline (speedup 1.0000x reference)
import functools

import jax
import jax.numpy as jnp
from jax import lax
from jax.experimental import pallas as pl
from jax.experimental.pallas import tpu as pltpu
from jax.experimental.pallas import tpu_sc as plsc

VOCAB = 100000
D = 128
B = 1024
L = 50

NC = 2
NS = 16
NW = NC * NS
LANES = 16

ITEMS_PER_W = B // NW
ROWS_PER_W = ITEMS_PER_W * L
CHUNK = 100
NCHUNK = ROWS_PER_W // CHUNK
ITEMS_PER_CORE = B // NC


def _sc_pool(x3, sidx3, table):
    mesh = plsc.VectorSubcoreMesh(core_axis_name="c", subcore_axis_name="s")

    @functools.partial(
        pl.kernel,
        out_type=jax.ShapeDtypeStruct((B, D), jnp.float32),
        mesh=mesh,
        scratch_types=[
            pltpu.VMEM((NCHUNK, CHUNK), jnp.int32),
            pltpu.VMEM((NCHUNK, CHUNK), jnp.int32),
            pltpu.VMEM((CHUNK, D), jnp.float32),
            pltpu.VMEM((CHUNK, D), jnp.float32),
            pltpu.VMEM((ITEMS_PER_W, D), jnp.float32),
            pltpu.VMEM_SHARED((ITEMS_PER_CORE, D), jnp.float32),
            pltpu.SemaphoreType.DMA,
            pltpu.SemaphoreType.DMA,
        ],
    )
    def pool_kernel(x_hbm, sidx_hbm, table_hbm, out_hbm,
                    idx_v, seg_v, rows_a, rows_b, pool_v, acc_sh,
                    sem_a, sem_b):
        c = lax.axis_index("c")
        s = lax.axis_index("s")
        wid = c * NS + s

        pltpu.sync_copy(x_hbm.at[wid], idx_v)
        pltpu.sync_copy(sidx_hbm.at[wid], seg_v)

        @pl.loop(0, ITEMS_PER_W)
        def _(i):
            @pl.loop(0, D, step=LANES)
            def _(col):
                pool_v[i, pl.ds(col, LANES)] = jnp.zeros((LANES,), jnp.float32)

        pltpu.sync_copy(pool_v, acc_sh.at[pl.ds(s * ITEMS_PER_W, ITEMS_PER_W)])
        plsc.subcore_barrier()

        @pl.loop(0, NCHUNK)
        def _(k):
            pltpu.async_copy(table_hbm.at[idx_v.at[k]], rows_a, sem_a).wait()
            pltpu.sync_copy(rows_a, acc_sh.at[seg_v.at[k]], add=True)

        plsc.subcore_barrier()

        pltpu.sync_copy(acc_sh.at[pl.ds(s * ITEMS_PER_W, ITEMS_PER_W)], pool_v)

        @pl.loop(0, ITEMS_PER_W)
        def _(i):
            @pl.loop(0, D, step=LANES)
            def _(col):
                pool_v[i, pl.ds(col, LANES)] = (
                    pool_v[i, pl.ds(col, LANES)] * (1.0 / L)
                )

        pltpu.sync_copy(pool_v, out_hbm.at[pl.ds(wid * ITEMS_PER_W, ITEMS_PER_W)])

    return pool_kernel(x3, sidx3, table)


TN = 2048
NT = pl.cdiv(VOCAB, TN)
TAIL = VOCAB - (NT - 1) * TN
NWB = 2
NOB = 4


def _tc_project(pooled, W, b2):
    mesh = pltpu.create_tensorcore_mesh("core")

    @functools.partial(
        pl.kernel,
        out_type=jax.ShapeDtypeStruct((B, VOCAB), jnp.float32),
        mesh=mesh,
        scratch_types=[
            pltpu.VMEM((B, D), jnp.float32),
            pltpu.VMEM((NWB, D, TN), jnp.float32),
            pltpu.VMEM((NOB, B, TN), jnp.float32),
            pltpu.VMEM((1, NT * TN), jnp.float32),
            pltpu.SemaphoreType.DMA,
            pltpu.SemaphoreType.DMA((NWB,)),
            pltpu.SemaphoreType.DMA((NOB,)),
        ],
    )
    def mm_kernel(p_hbm, w_hbm, b_hbm, o_hbm,
                  p_vmem, w_vmem, o_vmem, b_vmem, sem, wsems, osems):
        NF = NT - 1

        def start_w(i):
            return pltpu.async_copy(
                w_hbm.at[:, pl.ds(i * TN, TN)],
                w_vmem.at[i % NWB],
                wsems.at[i % NWB],
                priority=i % 2,
            )

        pltpu.async_copy(p_hbm, p_vmem, sem)
        w_copies = {i: start_w(i) for i in range(min(NWB, NF))}
        pltpu.async_copy(b_hbm, b_vmem, sem)
        pltpu.make_async_copy(p_hbm, p_vmem, sem).wait()
        pltpu.make_async_copy(b_hbm, b_vmem, sem).wait()

        o_copies = {}
        for i in range(NF):
            w_copies.pop(i).wait()
            if i >= NOB:
                o_copies.pop(i - NOB).wait()
            ob = i % NOB
            o_vmem[ob] = jnp.dot(
                p_vmem[...], w_vmem[i % NWB],
                preferred_element_type=jnp.float32,
            ) + b_vmem[0, pl.ds(i * TN, TN)]
            o_copies[i] = pltpu.async_copy(
                o_vmem.at[ob],
                o_hbm.at[:, pl.ds(i * TN, TN)],
                osems.at[ob],
                priority=i % 2,
            )
            if i + NWB < NF:
                w_copies[i + NWB] = start_w(i + NWB)

        def tail_step(w_t, b_t, o_t):
            o_t[...] = (
                jnp.dot(p_vmem[...], w_t[...],
                        preferred_element_type=jnp.float32)
                + b_t[...]
            )

        pltpu.emit_pipeline(
            tail_step,
            grid=(1,),
            in_specs=[
                pl.BlockSpec((D, TN), lambda i: (0, NF)),
                pl.BlockSpec((1, TN), lambda i: (0, NF)),
            ],
            out_specs=[pl.BlockSpec((B, TN), lambda i: (0, NF))],
        )(w_hbm, b_hbm, o_hbm)

        for i in sorted(o_copies):
            o_copies.pop(i).wait()

    return mm_kernel(pooled, W, b2)


def kernel(x, table, W, b):
    x3 = x.astype(jnp.int32).reshape(NW, NCHUNK, CHUNK)
    sidx3 = (
        (jnp.arange(B * L, dtype=jnp.int32) // L) % ITEMS_PER_CORE
    ).reshape(NW, NCHUNK, CHUNK)
    pooled = _sc_pool(x3, sidx3, table)
    b_pad = jnp.pad(b, (0, NT * TN - VOCAB)).reshape(1, NT * TN)
    return _tc_project(pooled, W, b_pad)


# speedup vs baseline: 1.0101x; 1.0101x over previous
import functools

import jax
import jax.numpy as jnp
from jax import lax
from jax.experimental import pallas as pl
from jax.experimental.pallas import tpu as pltpu
from jax.experimental.pallas import tpu_sc as plsc

VOCAB = 100000
D = 128
B = 1024
L = 50

NC = 2
NS = 16
NW = NC * NS
LANES = 16

ITEMS_PER_W = B // NW
ROWS_PER_W = ITEMS_PER_W * L
CHUNK = 100
NCHUNK = ROWS_PER_W // CHUNK
ITEMS_PER_CORE = B // NC


def _sc_pool(x3, sidx3, table):
    mesh = plsc.VectorSubcoreMesh(core_axis_name="c", subcore_axis_name="s")

    @functools.partial(
        pl.kernel,
        out_type=jax.ShapeDtypeStruct((B, D), jnp.float32),
        mesh=mesh,
        scratch_types=[
            pltpu.VMEM((NCHUNK, CHUNK), jnp.int32),
            pltpu.VMEM((NCHUNK, CHUNK), jnp.int32),
            pltpu.VMEM((CHUNK, D), jnp.float32),
            pltpu.VMEM((CHUNK, D), jnp.float32),
            pltpu.VMEM((ITEMS_PER_W, D), jnp.float32),
            pltpu.VMEM_SHARED((ITEMS_PER_CORE, D), jnp.float32),
            pltpu.SemaphoreType.DMA,
            pltpu.SemaphoreType.DMA,
        ],
    )
    def pool_kernel(x_hbm, sidx_hbm, table_hbm, out_hbm,
                    idx_v, seg_v, rows_a, rows_b, pool_v, acc_sh,
                    sem_a, sem_b):
        c = lax.axis_index("c")
        s = lax.axis_index("s")
        wid = c * NS + s

        pltpu.sync_copy(x_hbm.at[wid], idx_v)
        pltpu.sync_copy(sidx_hbm.at[wid], seg_v)

        @pl.loop(0, ITEMS_PER_W)
        def _(i):
            @pl.loop(0, D, step=LANES)
            def _(col):
                pool_v[i, pl.ds(col, LANES)] = jnp.zeros((LANES,), jnp.float32)

        pltpu.sync_copy(pool_v, acc_sh.at[pl.ds(s * ITEMS_PER_W, ITEMS_PER_W)])
        plsc.subcore_barrier()

        @pl.loop(0, NCHUNK)
        def _(k):
            pltpu.async_copy(table_hbm.at[idx_v.at[k]], rows_a, sem_a).wait()
            pltpu.sync_copy(rows_a, acc_sh.at[seg_v.at[k]], add=True)

        plsc.subcore_barrier()

        pltpu.sync_copy(acc_sh.at[pl.ds(s * ITEMS_PER_W, ITEMS_PER_W)], pool_v)

        @pl.loop(0, ITEMS_PER_W)
        def _(i):
            @pl.loop(0, D, step=LANES)
            def _(col):
                pool_v[i, pl.ds(col, LANES)] = (
                    pool_v[i, pl.ds(col, LANES)] * (1.0 / L)
                )

        pltpu.sync_copy(pool_v, out_hbm.at[pl.ds(wid * ITEMS_PER_W, ITEMS_PER_W)])

    return pool_kernel(x3, sidx3, table)


TN = 2048
NT = pl.cdiv(VOCAB, TN)
TAIL = VOCAB - (NT - 1) * TN
NWB = 2
NOB = 4


def _tc_project(pooled, W, b2):
    mesh = pltpu.create_tensorcore_mesh("core")

    @functools.partial(
        pl.kernel,
        out_type=jax.ShapeDtypeStruct((B, VOCAB), jnp.float32),
        mesh=mesh,
        scratch_types=[
            pltpu.VMEM((B, D), jnp.float32),
            pltpu.VMEM((NWB, D, TN), jnp.float32),
            pltpu.VMEM((NOB, B, TN), jnp.float32),
            pltpu.VMEM((1, NT * TN), jnp.float32),
            pltpu.SemaphoreType.DMA,
            pltpu.SemaphoreType.DMA((NWB,)),
            pltpu.SemaphoreType.DMA((NOB,)),
        ],
    )
    def mm_kernel(p_hbm, w_hbm, b_hbm, o_hbm,
                  p_vmem, w_vmem, o_vmem, b_vmem, sem, wsems, osems):
        NF = NT - 1

        def start_w(i):
            return pltpu.async_copy(
                w_hbm.at[:, pl.ds(i * TN, TN)],
                w_vmem.at[i % NWB],
                wsems.at[i % NWB],
                priority=i % 2,
            )

        pltpu.async_copy(p_hbm, p_vmem, sem)
        w_copies = {i: start_w(i) for i in range(min(NWB, NF))}
        pltpu.async_copy(b_hbm, b_vmem, sem)
        pltpu.make_async_copy(p_hbm, p_vmem, sem).wait()
        pltpu.make_async_copy(b_hbm, b_vmem, sem).wait()

        o_copies = {}
        for i in range(NF):
            w_copies.pop(i).wait()
            if i >= NOB:
                o_copies.pop(i - NOB).wait()
            ob = i % NOB
            o_vmem[ob] = jnp.broadcast_to(
                b_vmem[0, pl.ds(i * TN, TN)], (B, TN)
            )
            o_copies[i] = pltpu.async_copy(
                o_vmem.at[ob],
                o_hbm.at[:, pl.ds(i * TN, TN)],
                osems.at[ob],
                priority=i % 2,
            )
            if i + NWB < NF:
                w_copies[i + NWB] = start_w(i + NWB)

        def tail_step(w_t, b_t, o_t):
            o_t[...] = (
                jnp.dot(p_vmem[...], w_t[...],
                        preferred_element_type=jnp.float32)
                + b_t[...]
            )

        pltpu.emit_pipeline(
            tail_step,
            grid=(1,),
            in_specs=[
                pl.BlockSpec((D, TN), lambda i: (0, NF)),
                pl.BlockSpec((1, TN), lambda i: (0, NF)),
            ],
            out_specs=[pl.BlockSpec((B, TN), lambda i: (0, NF))],
        )(w_hbm, b_hbm, o_hbm)

        for i in sorted(o_copies):
            o_copies.pop(i).wait()

    return mm_kernel(pooled, W, b2)


def kernel(x, table, W, b):
    x3 = x.astype(jnp.int32).reshape(NW, NCHUNK, CHUNK)
    sidx3 = (
        (jnp.arange(B * L, dtype=jnp.int32) // L) % ITEMS_PER_CORE
    ).reshape(NW, NCHUNK, CHUNK)
    pooled = _sc_pool(x3, sidx3, table)
    b_pad = jnp.pad(b, (0, NT * TN - VOCAB)).reshape(1, NT * TN)
    return _tc_project(pooled, W, b_pad)


# speedup vs baseline: 1.0409x; 1.0305x over previous
import functools

import jax
import jax.numpy as jnp
from jax import lax
from jax.experimental import pallas as pl
from jax.experimental.pallas import tpu as pltpu
from jax.experimental.pallas import tpu_sc as plsc

VOCAB = 100000
D = 128
B = 1024
L = 50

NC = 2
NS = 16
NW = NC * NS
LANES = 16

ITEMS_PER_W = B // NW
ROWS_PER_W = ITEMS_PER_W * L
CHUNK = 100
NCHUNK = ROWS_PER_W // CHUNK
ITEMS_PER_CORE = B // NC


def _sc_pool(x3, sidx3, table):
    mesh = plsc.VectorSubcoreMesh(core_axis_name="c", subcore_axis_name="s")

    @functools.partial(
        pl.kernel,
        out_type=jax.ShapeDtypeStruct((B, D), jnp.float32),
        mesh=mesh,
        scratch_types=[
            pltpu.VMEM((NCHUNK, CHUNK), jnp.int32),
            pltpu.VMEM((NCHUNK, CHUNK), jnp.int32),
            pltpu.VMEM((CHUNK, D), jnp.float32),
            pltpu.VMEM((CHUNK, D), jnp.float32),
            pltpu.VMEM((ITEMS_PER_W, D), jnp.float32),
            pltpu.VMEM_SHARED((ITEMS_PER_CORE, D), jnp.float32),
            pltpu.SemaphoreType.DMA,
            pltpu.SemaphoreType.DMA,
        ],
    )
    def pool_kernel(x_hbm, sidx_hbm, table_hbm, out_hbm,
                    idx_v, seg_v, rows_a, rows_b, pool_v, acc_sh,
                    sem_a, sem_b):
        c = lax.axis_index("c")
        s = lax.axis_index("s")
        wid = c * NS + s

        pltpu.sync_copy(x_hbm.at[wid], idx_v)
        pltpu.sync_copy(sidx_hbm.at[wid], seg_v)

        @pl.loop(0, ITEMS_PER_W)
        def _(i):
            @pl.loop(0, D, step=LANES)
            def _(col):
                pool_v[i, pl.ds(col, LANES)] = jnp.zeros((LANES,), jnp.float32)

        pltpu.sync_copy(pool_v, acc_sh.at[pl.ds(s * ITEMS_PER_W, ITEMS_PER_W)])
        plsc.subcore_barrier()

        @pl.loop(0, NCHUNK)
        def _(k):
            pltpu.async_copy(table_hbm.at[idx_v.at[k]], rows_a, sem_a).wait()
            pltpu.sync_copy(rows_a, acc_sh.at[seg_v.at[k]], add=True)

        plsc.subcore_barrier()

        pltpu.sync_copy(acc_sh.at[pl.ds(s * ITEMS_PER_W, ITEMS_PER_W)], pool_v)

        @pl.loop(0, ITEMS_PER_W)
        def _(i):
            @pl.loop(0, D, step=LANES)
            def _(col):
                pool_v[i, pl.ds(col, LANES)] = (
                    pool_v[i, pl.ds(col, LANES)] * (1.0 / L)
                )

        pltpu.sync_copy(pool_v, out_hbm.at[pl.ds(wid * ITEMS_PER_W, ITEMS_PER_W)])

    return pool_kernel(x3, sidx3, table)


TN = 2048
NT = pl.cdiv(VOCAB, TN)
TAIL = VOCAB - (NT - 1) * TN
NWB = 2
NOB = 4


def _tc_project(pooled, W, b2):
    mesh = pltpu.create_tensorcore_mesh("core")

    @functools.partial(
        pl.kernel,
        out_type=jax.ShapeDtypeStruct((B, VOCAB), jnp.float32),
        mesh=mesh,
        scratch_types=[
            pltpu.VMEM((B, D), jnp.float32),
            pltpu.VMEM((NWB, D, TN), jnp.float32),
            pltpu.VMEM((NOB, B, TN), jnp.float32),
            pltpu.VMEM((1, NT * TN), jnp.float32),
            pltpu.SemaphoreType.DMA,
            pltpu.SemaphoreType.DMA((NWB,)),
            pltpu.SemaphoreType.DMA((NOB,)),
        ],
    )
    def mm_kernel(p_hbm, w_hbm, b_hbm, o_hbm,
                  p_vmem, w_vmem, o_vmem, b_vmem, sem, wsems, osems):
        NF = NT - 1

        def start_w(i):
            return pltpu.async_copy(
                w_hbm.at[:, pl.ds(i * TN, TN)],
                w_vmem.at[i % NWB],
                wsems.at[i % NWB],
                priority=i % 2,
            )

        pltpu.async_copy(p_hbm, p_vmem, sem)
        w_copies = {}
        pltpu.async_copy(b_hbm, b_vmem, sem)
        pltpu.make_async_copy(p_hbm, p_vmem, sem).wait()
        pltpu.make_async_copy(b_hbm, b_vmem, sem).wait()

        o_copies = {}
        for i in range(NF):
            if i >= NOB:
                o_copies.pop(i - NOB).wait()
            ob = i % NOB
            o_vmem[ob] = jnp.broadcast_to(
                b_vmem[0, pl.ds(i * TN, TN)], (B, TN)
            )
            o_copies[i] = pltpu.async_copy(
                o_vmem.at[ob],
                o_hbm.at[:, pl.ds(i * TN, TN)],
                osems.at[ob],
                priority=i % 2,
            )
        for i in sorted(o_copies):
            o_copies.pop(i).wait()

    return mm_kernel(pooled, W, b2)


def kernel(x, table, W, b):
    x3 = x.astype(jnp.int32).reshape(NW, NCHUNK, CHUNK)
    sidx3 = (
        (jnp.arange(B * L, dtype=jnp.int32) // L) % ITEMS_PER_CORE
    ).reshape(NW, NCHUNK, CHUNK)
    pooled = _sc_pool(x3, sidx3, table)
    b_pad = jnp.pad(b, (0, NT * TN - VOCAB)).reshape(1, NT * TN)
    return _tc_project(pooled, W, b_pad)


# speedup vs baseline: 2.8769x; 2.7638x over previous
import functools

import jax
import jax.numpy as jnp
from jax import lax
from jax.experimental import pallas as pl
from jax.experimental.pallas import tpu as pltpu
from jax.experimental.pallas import tpu_sc as plsc

VOCAB = 100000
D = 128
B = 1024
L = 50

NC = 2
NS = 16
NW = NC * NS
LANES = 16

ITEMS_PER_W = B // NW
ROWS_PER_W = ITEMS_PER_W * L
CHUNK = 100
NCHUNK = ROWS_PER_W // CHUNK
ITEMS_PER_CORE = B // NC


def _sc_pool(x3, sidx3, table):
    mesh = plsc.VectorSubcoreMesh(core_axis_name="c", subcore_axis_name="s")

    @functools.partial(
        pl.kernel,
        out_type=jax.ShapeDtypeStruct((B, D), jnp.float32),
        mesh=mesh,
        scratch_types=[
            pltpu.VMEM((NCHUNK, CHUNK), jnp.int32),
            pltpu.VMEM((NCHUNK, CHUNK), jnp.int32),
            pltpu.VMEM((CHUNK, D), jnp.float32),
            pltpu.VMEM((CHUNK, D), jnp.float32),
            pltpu.VMEM((ITEMS_PER_W, D), jnp.float32),
            pltpu.VMEM_SHARED((ITEMS_PER_CORE, D), jnp.float32),
            pltpu.SemaphoreType.DMA,
            pltpu.SemaphoreType.DMA,
        ],
    )
    def pool_kernel(x_hbm, sidx_hbm, table_hbm, out_hbm,
                    idx_v, seg_v, rows_a, rows_b, pool_v, acc_sh,
                    sem_a, sem_b):
        c = lax.axis_index("c")
        s = lax.axis_index("s")
        wid = c * NS + s

        pltpu.sync_copy(x_hbm.at[wid], idx_v)
        pltpu.sync_copy(sidx_hbm.at[wid], seg_v)

        @pl.loop(0, ITEMS_PER_W)
        def _(i):
            @pl.loop(0, D, step=LANES)
            def _(col):
                pool_v[i, pl.ds(col, LANES)] = jnp.zeros((LANES,), jnp.float32)

        pltpu.sync_copy(pool_v, acc_sh.at[pl.ds(s * ITEMS_PER_W, ITEMS_PER_W)])
        plsc.subcore_barrier()

        @pl.loop(0, NCHUNK)
        def _(k):
            pltpu.async_copy(table_hbm.at[idx_v.at[k]], rows_a, sem_a).wait()
            pltpu.sync_copy(rows_a, acc_sh.at[seg_v.at[k]], add=True)

        plsc.subcore_barrier()

        pltpu.sync_copy(acc_sh.at[pl.ds(s * ITEMS_PER_W, ITEMS_PER_W)], pool_v)

        @pl.loop(0, ITEMS_PER_W)
        def _(i):
            @pl.loop(0, D, step=LANES)
            def _(col):
                pool_v[i, pl.ds(col, LANES)] = (
                    pool_v[i, pl.ds(col, LANES)] * (1.0 / L)
                )

        pltpu.sync_copy(pool_v, out_hbm.at[pl.ds(wid * ITEMS_PER_W, ITEMS_PER_W)])

    return pool_kernel(x3, sidx3, table)


TN = 2048
NT = pl.cdiv(VOCAB, TN)
TAIL = VOCAB - (NT - 1) * TN
NWB = 2
NOB = 4


def _tc_project_t(pooled, Wt, b2):
    mesh = pltpu.create_tensorcore_mesh("core")

    @functools.partial(
        pl.kernel,
        out_type=jax.ShapeDtypeStruct((VOCAB, B), jnp.float32),
        mesh=mesh,
        scratch_types=[
            pltpu.VMEM((B, D), jnp.float32),
            pltpu.VMEM((D, B), jnp.float32),
            pltpu.VMEM((NWB, TN, D), jnp.float32),
            pltpu.VMEM((NOB, TN, B), jnp.float32),
            pltpu.VMEM((1, NT * TN), jnp.float32),
            pltpu.SemaphoreType.DMA,
            pltpu.SemaphoreType.DMA((NWB,)),
            pltpu.SemaphoreType.DMA((NOB,)),
        ],
    )
    def mm_kernel(p_hbm, w_hbm, b_hbm, o_hbm,
                  p_vmem, pt_vmem, w_vmem, o_vmem, b_vmem, sem, wsems, osems):
        def rows(i):
            return TN if i < NT - 1 else TAIL

        def start_w(i):
            n = rows(i)
            return pltpu.async_copy(
                w_hbm.at[pl.ds(i * TN, n)],
                w_vmem.at[i % NWB, pl.ds(0, n)],
                wsems.at[i % NWB],
                priority=i % 2,
            )

        pltpu.async_copy(p_hbm, p_vmem, sem)
        w_copies = {i: start_w(i) for i in range(min(NWB, NT))}
        pltpu.async_copy(b_hbm, b_vmem, sem)
        pltpu.make_async_copy(p_hbm, p_vmem, sem).wait()
        pt_vmem[...] = p_vmem[...].T
        pltpu.make_async_copy(b_hbm, b_vmem, sem).wait()

        o_copies = {}
        for i in range(NT):
            w_copies.pop(i).wait()
            if i >= NOB:
                o_copies.pop(i - NOB).wait()
            n = rows(i)
            ob = i % NOB
            o_vmem[ob, pl.ds(0, n)] = jnp.dot(
                w_vmem[i % NWB, pl.ds(0, n)], pt_vmem[...],
                preferred_element_type=jnp.float32,
            ) + b_vmem[0, pl.ds(i * TN, n)].reshape(n, 1)
            o_copies[i] = pltpu.async_copy(
                o_vmem.at[ob, pl.ds(0, n)],
                o_hbm.at[pl.ds(i * TN, n)],
                osems.at[ob],
                priority=i % 2,
            )
            if i + NWB < NT:
                w_copies[i + NWB] = start_w(i + NWB)
        for i in sorted(o_copies):
            o_copies.pop(i).wait()

    return mm_kernel(pooled, Wt, b2)


def kernel(x, table, W, b):
    x3 = x.astype(jnp.int32).reshape(NW, NCHUNK, CHUNK)
    sidx3 = (
        (jnp.arange(B * L, dtype=jnp.int32) // L) % ITEMS_PER_CORE
    ).reshape(NW, NCHUNK, CHUNK)
    pooled = _sc_pool(x3, sidx3, table)
    b_pad = jnp.pad(b, (0, NT * TN - VOCAB)).reshape(1, NT * TN)
    out_t = _tc_project_t(pooled, W.T, b_pad)
    return out_t.T


# speedup vs baseline: 2.9775x; 1.0349x over previous
import functools

import jax
import jax.numpy as jnp
import numpy as np
from jax import lax
from jax.experimental import pallas as pl
from jax.experimental.pallas import tpu as pltpu
from jax.experimental.pallas import tpu_sc as plsc

VOCAB = 100000
D = 128
B = 1024
L = 50

NC = 2
NS = 16
NW = NC * NS
LANES = 16

ITEMS_PER_W = B // NW
ROWS_PER_W = ITEMS_PER_W * L
CHUNK = 100
NCHUNK = ROWS_PER_W // CHUNK
ITEMS_PER_CORE = B // NC


def _sc_pool(x3, sidx3, table):
    mesh = plsc.VectorSubcoreMesh(core_axis_name="c", subcore_axis_name="s")

    @functools.partial(
        pl.kernel,
        out_type=jax.ShapeDtypeStruct((B, D), jnp.float32),
        mesh=mesh,
        scratch_types=[
            pltpu.VMEM((NCHUNK, CHUNK), jnp.int32),
            pltpu.VMEM((NCHUNK, CHUNK), jnp.int32),
            pltpu.VMEM((CHUNK, D), jnp.float32),
            pltpu.VMEM((CHUNK, D), jnp.float32),
            pltpu.VMEM((ITEMS_PER_W, D), jnp.float32),
            pltpu.VMEM_SHARED((ITEMS_PER_CORE, D), jnp.float32),
            pltpu.SemaphoreType.DMA,
            pltpu.SemaphoreType.DMA,
        ],
    )
    def pool_kernel(x_hbm, sidx_hbm, table_hbm, out_hbm,
                    idx_v, seg_v, rows_a, rows_b, pool_v, acc_sh,
                    sem_a, sem_b):
        c = lax.axis_index("c")
        s = lax.axis_index("s")
        wid = c * NS + s

        pltpu.sync_copy(x_hbm.at[wid], idx_v)
        pltpu.sync_copy(sidx_hbm.at[wid], seg_v)

        @pl.loop(0, ITEMS_PER_W)
        def _(i):
            @pl.loop(0, D, step=LANES)
            def _(col):
                pool_v[i, pl.ds(col, LANES)] = jnp.zeros((LANES,), jnp.float32)

        pltpu.sync_copy(pool_v, acc_sh.at[pl.ds(s * ITEMS_PER_W, ITEMS_PER_W)])
        plsc.subcore_barrier()

        pltpu.async_copy(table_hbm.at[idx_v.at[0]], rows_a, sem_a)

        @pl.loop(0, NCHUNK, step=2)
        def _(k):
            pltpu.make_async_copy(table_hbm.at[idx_v.at[k]], rows_a, sem_a).wait()
            pltpu.async_copy(table_hbm.at[idx_v.at[k + 1]], rows_b, sem_b)
            pltpu.sync_copy(rows_a, acc_sh.at[seg_v.at[k]], add=True)

            pltpu.make_async_copy(
                table_hbm.at[idx_v.at[k + 1]], rows_b, sem_b).wait()

            @pl.when(k + 2 < NCHUNK)
            def _():
                pltpu.async_copy(table_hbm.at[idx_v.at[k + 2]], rows_a, sem_a)

            pltpu.sync_copy(rows_b, acc_sh.at[seg_v.at[k + 1]], add=True)

        plsc.subcore_barrier()

        pltpu.sync_copy(acc_sh.at[pl.ds(s * ITEMS_PER_W, ITEMS_PER_W)], pool_v)

        @pl.loop(0, ITEMS_PER_W)
        def _(i):
            @pl.loop(0, D, step=LANES)
            def _(col):
                pool_v[i, pl.ds(col, LANES)] = (
                    pool_v[i, pl.ds(col, LANES)] * (1.0 / L)
                )

        pltpu.sync_copy(pool_v, out_hbm.at[pl.ds(wid * ITEMS_PER_W, ITEMS_PER_W)])

    return pool_kernel(x3, sidx3, table)


TN = 2048
NT = pl.cdiv(VOCAB, TN)
TAIL = VOCAB - (NT - 1) * TN
NWB = 2
NOB = 4


def _tc_project_t(pooled, Wt, b2):
    mesh = pltpu.create_tensorcore_mesh("core")

    @functools.partial(
        pl.kernel,
        out_type=jax.ShapeDtypeStruct((VOCAB, B), jnp.float32),
        mesh=mesh,
        scratch_types=[
            pltpu.VMEM((B, D), jnp.float32),
            pltpu.VMEM((D, B), jnp.float32),
            pltpu.VMEM((NWB, TN, D), jnp.float32),
            pltpu.VMEM((NOB, TN, B), jnp.float32),
            pltpu.VMEM((1, NT * TN), jnp.float32),
            pltpu.SemaphoreType.DMA,
            pltpu.SemaphoreType.DMA((NWB,)),
            pltpu.SemaphoreType.DMA((NOB,)),
        ],
    )
    def mm_kernel(p_hbm, w_hbm, b_hbm, o_hbm,
                  p_vmem, pt_vmem, w_vmem, o_vmem, b_vmem, sem, wsems, osems):
        def rows(i):
            return TN if i < NT - 1 else TAIL

        def start_w(i):
            n = rows(i)
            return pltpu.async_copy(
                w_hbm.at[pl.ds(i * TN, n)],
                w_vmem.at[i % NWB, pl.ds(0, n)],
                wsems.at[i % NWB],
                priority=i % 2,
            )

        pltpu.async_copy(p_hbm, p_vmem, sem)
        w_copies = {i: start_w(i) for i in range(min(NWB, NT))}
        pltpu.async_copy(b_hbm, b_vmem, sem)
        pltpu.make_async_copy(p_hbm, p_vmem, sem).wait()
        pt_vmem[...] = p_vmem[...].T
        pltpu.make_async_copy(b_hbm, b_vmem, sem).wait()

        o_copies = {}
        for i in range(NT):
            w_copies.pop(i).wait()
            if i >= NOB:
                o_copies.pop(i - NOB).wait()
            n = rows(i)
            ob = i % NOB
            o_vmem[ob, pl.ds(0, n)] = jnp.dot(
                w_vmem[i % NWB, pl.ds(0, n)], pt_vmem[...],
                preferred_element_type=jnp.float32,
            ) + b_vmem[0, pl.ds(i * TN, n)].reshape(n, 1)
            o_copies[i] = pltpu.async_copy(
                o_vmem.at[ob, pl.ds(0, n)],
                o_hbm.at[pl.ds(i * TN, n)],
                osems.at[ob],
                priority=i % 2,
            )
            if i + NWB < NT:
                w_copies[i + NWB] = start_w(i + NWB)
        for i in sorted(o_copies):
            o_copies.pop(i).wait()

    return mm_kernel(pooled, Wt, b2)


_SIDX3 = (
    (np.arange(B * L, dtype=np.int32) // L) % ITEMS_PER_CORE
).reshape(NW, NCHUNK, CHUNK)


def kernel(x, table, W, b):
    x3 = x.astype(jnp.int32).reshape(NW, NCHUNK, CHUNK)
    sidx3 = jnp.asarray(_SIDX3)
    pooled = _sc_pool(x3, sidx3, table)
    b_pad = jnp.pad(b, (0, NT * TN - VOCAB)).reshape(1, NT * TN)
    out_t = _tc_project_t(pooled, W.T, b_pad)
    return out_t.T


# speedup vs baseline: 2.9807x; 1.0011x over previous
import functools

import jax
import jax.numpy as jnp
import numpy as np
from jax import lax
from jax.experimental import pallas as pl
from jax.experimental.pallas import tpu as pltpu
from jax.experimental.pallas import tpu_sc as plsc

VOCAB = 100000
D = 128
B = 1024
L = 50

NC = 2
NS = 16
NW = NC * NS
LANES = 16

ITEMS_PER_W = B // NW
ROWS_PER_W = ITEMS_PER_W * L
CHUNK = 100
NCHUNK = ROWS_PER_W // CHUNK
ITEMS_PER_CORE = B // NC


def _sc_pool(x3, sidx3, table):
    mesh = plsc.VectorSubcoreMesh(core_axis_name="c", subcore_axis_name="s")

    @functools.partial(
        pl.kernel,
        out_type=jax.ShapeDtypeStruct((B, D), jnp.float32),
        mesh=mesh,
        scratch_types=[
            pltpu.VMEM((NCHUNK, CHUNK), jnp.int32),
            pltpu.VMEM((NCHUNK, CHUNK), jnp.int32),
            pltpu.VMEM((CHUNK, D), jnp.float32),
            pltpu.VMEM((CHUNK, D), jnp.float32),
            pltpu.VMEM((ITEMS_PER_W, D), jnp.float32),
            pltpu.VMEM_SHARED((ITEMS_PER_CORE, D), jnp.float32),
            pltpu.SemaphoreType.DMA,
            pltpu.SemaphoreType.DMA,
        ],
    )
    def pool_kernel(x_hbm, sidx_hbm, table_hbm, out_hbm,
                    idx_v, seg_v, rows_a, rows_b, pool_v, acc_sh,
                    sem_a, sem_b):
        c = lax.axis_index("c")
        s = lax.axis_index("s")
        wid = c * NS + s

        pltpu.sync_copy(x_hbm.at[wid], idx_v)
        pltpu.sync_copy(sidx_hbm.at[wid], seg_v)

        @pl.loop(0, ITEMS_PER_W)
        def _(i):
            @pl.loop(0, D, step=LANES)
            def _(col):
                pool_v[i, pl.ds(col, LANES)] = jnp.zeros((LANES,), jnp.float32)

        pltpu.sync_copy(pool_v, acc_sh.at[pl.ds(s * ITEMS_PER_W, ITEMS_PER_W)])
        plsc.subcore_barrier()

        pltpu.async_copy(table_hbm.at[idx_v.at[0]], rows_a, sem_a)

        @pl.loop(0, NCHUNK, step=2)
        def _(k):
            pltpu.make_async_copy(table_hbm.at[idx_v.at[k]], rows_a, sem_a).wait()
            pltpu.async_copy(table_hbm.at[idx_v.at[k + 1]], rows_b, sem_b)
            pltpu.sync_copy(rows_a, acc_sh.at[seg_v.at[k]], add=True)

            pltpu.make_async_copy(
                table_hbm.at[idx_v.at[k + 1]], rows_b, sem_b).wait()

            @pl.when(k + 2 < NCHUNK)
            def _():
                pltpu.async_copy(table_hbm.at[idx_v.at[k + 2]], rows_a, sem_a)

            pltpu.sync_copy(rows_b, acc_sh.at[seg_v.at[k + 1]], add=True)

        plsc.subcore_barrier()

        pltpu.sync_copy(acc_sh.at[pl.ds(s * ITEMS_PER_W, ITEMS_PER_W)], pool_v)

        @pl.loop(0, ITEMS_PER_W)
        def _(i):
            @pl.loop(0, D, step=LANES)
            def _(col):
                pool_v[i, pl.ds(col, LANES)] = (
                    pool_v[i, pl.ds(col, LANES)] * (1.0 / L)
                )

        pltpu.sync_copy(pool_v, out_hbm.at[pl.ds(wid * ITEMS_PER_W, ITEMS_PER_W)])

    return pool_kernel(x3, sidx3, table)


TN = 2048
NT = pl.cdiv(VOCAB, TN)
TAIL = VOCAB - (NT - 1) * TN
NWB = 2
NOB = 6


def _tc_project_t(pooled, Wt, b2):
    mesh = pltpu.create_tensorcore_mesh("core")

    @functools.partial(
        pl.kernel,
        out_type=jax.ShapeDtypeStruct((VOCAB, B), jnp.float32),
        mesh=mesh,
        scratch_types=[
            pltpu.VMEM((B, D), jnp.float32),
            pltpu.VMEM((D, B), jnp.float32),
            pltpu.VMEM((NWB, TN, D), jnp.float32),
            pltpu.VMEM((NOB, TN, B), jnp.float32),
            pltpu.VMEM((1, NT * TN), jnp.float32),
            pltpu.SemaphoreType.DMA,
            pltpu.SemaphoreType.DMA((NWB,)),
            pltpu.SemaphoreType.DMA((NOB,)),
        ],
    )
    def mm_kernel(p_hbm, w_hbm, b_hbm, o_hbm,
                  p_vmem, pt_vmem, w_vmem, o_vmem, b_vmem, sem, wsems, osems):
        def rows(i):
            return TN if i < NT - 1 else TAIL

        def start_w(i):
            n = rows(i)
            return pltpu.async_copy(
                w_hbm.at[pl.ds(i * TN, n)],
                w_vmem.at[i % NWB, pl.ds(0, n)],
                wsems.at[i % NWB],
                priority=i % 2,
            )

        pltpu.async_copy(p_hbm, p_vmem, sem)
        w_copies = {i: start_w(i) for i in range(min(NWB, NT))}
        pltpu.async_copy(b_hbm, b_vmem, sem)
        pltpu.make_async_copy(p_hbm, p_vmem, sem).wait()
        pt_vmem[...] = p_vmem[...].T
        pltpu.make_async_copy(b_hbm, b_vmem, sem).wait()

        o_copies = {}
        for i in range(NT):
            w_copies.pop(i).wait()
            if i >= NOB:
                o_copies.pop(i - NOB).wait()
            n = rows(i)
            ob = i % NOB
            o_vmem[ob, pl.ds(0, n)] = jnp.dot(
                w_vmem[i % NWB, pl.ds(0, n)], pt_vmem[...],
                preferred_element_type=jnp.float32,
            ) + b_vmem[0, pl.ds(i * TN, n)].reshape(n, 1)
            o_copies[i] = pltpu.async_copy(
                o_vmem.at[ob, pl.ds(0, n)],
                o_hbm.at[pl.ds(i * TN, n)],
                osems.at[ob],
                priority=i % 2,
            )
            if i + NWB < NT:
                w_copies[i + NWB] = start_w(i + NWB)
        for i in sorted(o_copies):
            o_copies.pop(i).wait()

    return mm_kernel(pooled, Wt, b2)


_SIDX3 = (
    (np.arange(B * L, dtype=np.int32) // L) % ITEMS_PER_CORE
).reshape(NW, NCHUNK, CHUNK)


def kernel(x, table, W, b):
    x3 = x.astype(jnp.int32).reshape(NW, NCHUNK, CHUNK)
    sidx3 = jnp.asarray(_SIDX3)
    pooled = _sc_pool(x3, sidx3, table)
    b_pad = jnp.pad(b, (0, NT * TN - VOCAB)).reshape(1, NT * TN)
    out_t = _tc_project_t(pooled, W.T, b_pad)
    return out_t.T
